# baseline (speedup 1.0000x reference)
import functools

import jax
import jax.numpy as jnp
from jax import lax
from jax.experimental import pallas as pl
from jax.experimental.pallas import tpu as pltpu
from jax.experimental.pallas import tpu_sc as plsc

NUM_CHARS = 1000
BATCH = 4096
HIST = 50
B = BATCH * HIST
NC = 2
NS = 16
NW = NC * NS
BPW = B // NW
C = 16
NCHUNK = BPW // C
NBUF = 3
NGROUP = NCHUNK // NBUF


@functools.partial(
    pl.kernel,
    mesh=plsc.VectorSubcoreMesh(core_axis_name="c", subcore_axis_name="s"),
    out_type=jax.ShapeDtypeStruct((B, NUM_CHARS), jnp.float32),
    scratch_types=(
        [pltpu.VMEM((BPW,), jnp.int32)]
        + [pltpu.VMEM((C, NUM_CHARS), jnp.float32) for _ in range(NBUF)]
        + [pltpu.SemaphoreType.DMA for _ in range(2 * NBUF)]
    ),
    compiler_params=pltpu.CompilerParams(use_tc_tiling_on_sc=False),
)
def _emb_gather(idx_hbm, table_hbm, out_hbm, idx_v, *bufs_and_sems):
    rows = bufs_and_sems[:NBUF]
    gsem = bufs_and_sems[NBUF:2 * NBUF]
    ssem = bufs_and_sems[2 * NBUF:]

    wid = lax.axis_index("s") * NC + lax.axis_index("c")
    base = wid * BPW
    pltpu.sync_copy(idx_hbm.at[pl.ds(base, BPW)], idx_v)

    def chunk(b, off, drain_scatter):
        dst = out_hbm.at[pl.ds(base + off, C)]
        if drain_scatter:
            pltpu.make_async_copy(rows[b], dst, ssem[b]).wait()
        pltpu.async_copy(
            table_hbm.at[idx_v.at[pl.ds(off, C)]], rows[b], gsem[b]
        ).wait()
        pltpu.async_copy(rows[b], dst, ssem[b])

    for b in range(NBUF):
        chunk(b, b * C, drain_scatter=False)

    def group(go, carry):
        for b in range(NBUF):
            chunk(b, (go * NBUF + b) * C, drain_scatter=True)
        return carry

    lax.fori_loop(1, NGROUP, group, 0)

    for b in range(NBUF):
        off = (NGROUP - 1) * NBUF * C + b * C
        pltpu.make_async_copy(
            rows[b], out_hbm.at[pl.ds(base + off, C)], ssem[b]
        ).wait()


def kernel(indices, emb_weight):
    idx_flat = indices.reshape(-1).astype(jnp.int32)
    out = _emb_gather(idx_flat, emb_weight)
    return out.reshape(BATCH, HIST, NUM_CHARS)


# speedup vs baseline: 1.0513x; 1.0513x over previous
import functools

import jax
import jax.numpy as jnp
from jax import lax
from jax.experimental import pallas as pl
from jax.experimental.pallas import tpu as pltpu
from jax.experimental.pallas import tpu_sc as plsc

NUM_CHARS = 1000
BATCH = 4096
HIST = 50
B = BATCH * HIST
NC = 2
NS = 16
NW = NC * NS
BPW = BATCH // NW
NBUF = 2
HP = 56


@functools.partial(
    pl.kernel,
    mesh=plsc.VectorSubcoreMesh(core_axis_name="c", subcore_axis_name="s"),
    out_type=jax.ShapeDtypeStruct((BATCH, HIST, NUM_CHARS), jnp.float32),
    scratch_types=(
        [pltpu.VMEM((BPW * HP,), jnp.int32)]
        + [pltpu.VMEM((HIST, NUM_CHARS), jnp.float32) for _ in range(NBUF)]
        + [pltpu.SemaphoreType.DMA for _ in range(2 * NBUF)]
    ),
    compiler_params=pltpu.CompilerParams(use_tc_tiling_on_sc=False),
)
def _emb_gather(idx_hbm, table_hbm, out_hbm, idx_v, *bufs_and_sems):
    rows = bufs_and_sems[:NBUF]
    gsem = bufs_and_sems[NBUF:2 * NBUF]
    ssem = bufs_and_sems[2 * NBUF:]

    wid = lax.axis_index("s") * NC + lax.axis_index("c")
    base = wid * BPW
    pltpu.sync_copy(idx_hbm.at[pl.ds(base * HP, BPW * HP)], idx_v)

    def chunk(b, g, drain_scatter):
        dst = out_hbm.at[base + g]
        if drain_scatter:
            pltpu.make_async_copy(rows[b], dst, ssem[b]).wait()
        pltpu.async_copy(
            table_hbm.at[idx_v.at[pl.ds(g * HP, HIST)]], rows[b], gsem[b]
        ).wait()
        pltpu.async_copy(rows[b], dst, ssem[b])

    for b in range(NBUF):
        chunk(b, b, drain_scatter=False)

    def group(go, carry):
        for b in range(NBUF):
            chunk(b, go * NBUF + b, drain_scatter=True)
        return carry

    lax.fori_loop(1, BPW // NBUF, group, 0)

    for b in range(NBUF):
        g = BPW - NBUF + b
        pltpu.make_async_copy(rows[b], out_hbm.at[base + g], ssem[b]).wait()


def kernel(indices, emb_weight):
    idx_pad = jnp.pad(indices.astype(jnp.int32), ((0, 0), (0, HP - HIST)))
    return _emb_gather(idx_pad.reshape(-1), emb_weight)
